# Initial kernel scaffold; baseline (speedup 1.0000x reference)
import jax
import jax.numpy as jnp
from jax.experimental import pallas as pl


def kernel(x, edge_index, batch, edge_attr, node_emb, edge_W, edge_b, vn_emb, conv_eps, conv_lin1_W, conv_lin1_b, conv_bn_g, conv_bn_b, conv_lin2_W, conv_lin2_b, bn_g, bn_b, vn_lin1_W, vn_lin1_b, vn_bn1_g, vn_bn1_b, vn_lin2_W, vn_lin2_b, vn_bn2_g, vn_bn2_b, cls_W1, cls_b1, cls_W2, cls_b2):
    raise NotImplementedError("write your pallas kernel here")


# speedup vs baseline: 2.2797x; 2.2797x over previous
import jax
import jax.numpy as jnp
from jax import lax
from jax.experimental import pallas as pl
from jax.experimental.pallas import tpu as pltpu
from jax.experimental.pallas import tpu_sc as plsc

N_NODES = 10000
N_EDGES = 320000
EMB = 128
EDGE_DIM = 16
NUM_FEATURES = 128
NUM_CLASSES = 6
NUM_LAYERS = 5
NUM_GRAPHS = 64

NC = 2
NS = 16
NW = NC * NS
CHUNK = 128
EPW = 10112
NCHUNK = EPW // CHUNK
E_PAD = NW * EPW
ROWS_PER_TILE = 632
AGG_ROWS = NS * ROWS_PER_TILE
DUMMY_ROW = N_NODES

F32 = jnp.float32


def _edge_sc_body(hp_hbm, srcs_hbm, dsts_hbm, ea_hbm, zero_hbm, out_hbm,
                  idx_s, idx_d, rows, eabuf, aggr_sh, sem):
    c = lax.axis_index("c")
    s = lax.axis_index("s")
    w = s * NC + c
    pltpu.sync_copy(zero_hbm, aggr_sh.at[pl.ds(s * ROWS_PER_TILE, ROWS_PER_TILE)])
    plsc.subcore_barrier()

    def chunk_body(j, carry):
        pltpu.sync_copy(srcs_hbm.at[w, j], idx_s.at[0])
        pltpu.sync_copy(dsts_hbm.at[w, j], idx_d.at[0])
        pltpu.async_copy(hp_hbm.at[idx_s.at[0]], rows, sem).wait()
        pltpu.sync_copy(ea_hbm.at[w, j], eabuf)

        def row_body(r, carry2):
            for g in range(EMB // 16):
                sl = pl.ds(g * 16, 16)
                rows[r, sl] = jnp.maximum(rows[r, sl] + eabuf[r, sl], 0.0)
            return carry2

        lax.fori_loop(0, CHUNK, row_body, 0)
        pltpu.sync_copy(rows, aggr_sh.at[idx_d.at[0]], add=True)
        return carry

    lax.fori_loop(0, NCHUNK, chunk_body, 0)
    plsc.subcore_barrier()
    pltpu.sync_copy(aggr_sh.at[pl.ds(s * ROWS_PER_TILE, ROWS_PER_TILE)],
                    out_hbm.at[c, pl.ds(s * ROWS_PER_TILE, ROWS_PER_TILE)])


_edge_sc = pl.kernel(
    _edge_sc_body,
    out_type=jax.ShapeDtypeStruct((NC, AGG_ROWS, EMB), F32),
    mesh=plsc.VectorSubcoreMesh(core_axis_name="c", subcore_axis_name="s",
                                num_cores=NC, num_subcores=NS),
    scratch_types=[
        pltpu.VMEM((1, CHUNK), jnp.int32),
        pltpu.VMEM((1, CHUNK), jnp.int32),
        pltpu.VMEM((CHUNK, EMB), F32),
        pltpu.VMEM((CHUNK, EMB), F32),
        pltpu.VMEM_SHARED((AGG_ROWS, EMB), F32),
        pltpu.SemaphoreType.DMA,
    ],
)


def _prep_body(x_ref, emb_ref, vne_ref, out_ref):
    oh = (lax.broadcasted_iota(jnp.int32, (N_NODES, NUM_FEATURES), 1)
          == x_ref[...]).astype(F32)
    out_ref[...] = (jnp.dot(oh, emb_ref[...], preferred_element_type=F32, precision=lax.Precision.HIGHEST)
                    + vne_ref[...])


_prep = pl.pallas_call(
    _prep_body,
    out_shape=jax.ShapeDtypeStruct((N_NODES, EMB), F32),
)


def _ea_body(attr_ref, w_ref, b_ref, out_ref):
    out_ref[...] = (jnp.dot(attr_ref[...], w_ref[...], preferred_element_type=F32)
                    + b_ref[...])


_ea = pl.pallas_call(
    _ea_body,
    grid=(NW,),
    in_specs=[pl.BlockSpec((EPW, EDGE_DIM), lambda i: (i, 0)),
              pl.BlockSpec((EDGE_DIM, EMB), lambda i: (0, 0)),
              pl.BlockSpec((1, EMB), lambda i: (0, 0))],
    out_specs=pl.BlockSpec((EPW, EMB), lambda i: (i, 0)),
    out_shape=jax.ShapeDtypeStruct((E_PAD, EMB), F32),
)


def _bn_rows(h, g, b):
    mean = jnp.mean(h, axis=0)
    var = jnp.var(h, axis=0)
    return g * (h - mean) / jnp.sqrt(var + 1e-5) + b


def _bn(t, g, b):
    mean = jnp.mean(t, axis=0, keepdims=True)
    var = jnp.mean((t - mean) ** 2, axis=0, keepdims=True)
    return g * (t - mean) * lax.rsqrt(var + 1e-5) + b


def _layer_core(hp_ref, parts_ref, eps, w1, b1, g1, bb1, w2, b2, g2, bb2):
    agg = parts_ref[0, :N_NODES, :] + parts_ref[1, :N_NODES, :]
    t = (1.0 + eps) * hp_ref[...] + agg
    t = jnp.dot(t, w1, preferred_element_type=F32) + b1
    t = jnp.maximum(_bn(t, g1, bb1), 0.0)
    t = jnp.dot(t, w2, preferred_element_type=F32) + b2
    return jnp.maximum(_bn(t, g2, bb2), 0.0)


def _conv_body(hp_ref, parts_ref, eps_ref,
               w1_ref, b1_ref, g1_ref, bb1_ref, w2_ref, b2_ref, g2_ref, bb2_ref,
               h_ref):
    h_ref[...] = _layer_core(hp_ref, parts_ref, eps_ref[0, 0],
                             w1_ref[...], b1_ref[...], g1_ref[...], bb1_ref[...],
                             w2_ref[...], b2_ref[...], g2_ref[...], bb2_ref[...])


_conv = pl.pallas_call(
    _conv_body,
    out_shape=jax.ShapeDtypeStruct((N_NODES, EMB), F32),
)


def _cls_body(gemb_ref, cw1_ref, cb1_ref, cw2_ref, cb2_ref, logits_ref):
    hid = jnp.maximum(
        jnp.dot(gemb_ref[...], cw1_ref[...], preferred_element_type=F32)
        + cb1_ref[...], 0.0)
    logits_ref[...] = (jnp.dot(hid, cw2_ref[...], preferred_element_type=F32)
                       + cb2_ref[...])


_cls = pl.pallas_call(
    _cls_body,
    out_shape=jax.ShapeDtypeStruct((NUM_GRAPHS, NUM_CLASSES), F32),
)


def kernel(x, edge_index, batch, edge_attr, node_emb, edge_W, edge_b, vn_emb,
           conv_eps, conv_lin1_W, conv_lin1_b, conv_bn_g, conv_bn_b,
           conv_lin2_W, conv_lin2_b, bn_g, bn_b,
           vn_lin1_W, vn_lin1_b, vn_bn1_g, vn_bn1_b,
           vn_lin2_W, vn_lin2_b, vn_bn2_g, vn_bn2_b,
           cls_W1, cls_b1, cls_W2, cls_b2):
    x2 = x.reshape(N_NODES, 1).astype(jnp.int32)
    bcol = batch.reshape(N_NODES, 1).astype(jnp.int32)
    brow = batch.reshape(1, N_NODES).astype(jnp.int32)
    src = edge_index[0].astype(jnp.int32)
    dst = edge_index[1].astype(jnp.int32)
    pad = E_PAD - N_EDGES
    srcs = jnp.concatenate([src, jnp.zeros((pad,), jnp.int32)]).reshape(NW, NCHUNK, CHUNK)
    dsts = jnp.concatenate([dst, jnp.full((pad,), DUMMY_ROW, jnp.int32)]).reshape(NW, NCHUNK, CHUNK)
    attr_pad = jnp.concatenate([edge_attr, jnp.zeros((pad, EDGE_DIM), F32)], axis=0)
    zero_blk = jnp.zeros((ROWS_PER_TILE, EMB), F32)
    eb = edge_b.reshape(1, EMB)
    vne = vn_emb.reshape(1, EMB)

    hp = _prep(x2, node_emb, vne)
    ea = _ea(attr_pad, edge_W, eb).reshape(NW, NCHUNK, CHUNK, EMB)
    vn = vn_emb[jnp.zeros((NUM_GRAPHS,), dtype=jnp.int32)]

    for i in range(NUM_LAYERS):
        parts = _edge_sc(hp, srcs, dsts, ea, zero_blk)
        eps_i = conv_eps[i].reshape(1, 1)
        h = _conv(hp, parts, eps_i,
                  conv_lin1_W[i], conv_lin1_b[i].reshape(1, EMB),
                  conv_bn_g[i].reshape(1, EMB), conv_bn_b[i].reshape(1, EMB),
                  conv_lin2_W[i], conv_lin2_b[i].reshape(1, EMB),
                  bn_g[i].reshape(1, EMB), bn_b[i].reshape(1, EMB))
        if i < NUM_LAYERS - 1:
            pooled = jax.ops.segment_sum(h, batch, num_segments=NUM_GRAPHS)
            tmp = vn + pooled
            u = tmp @ vn_lin1_W[i] + vn_lin1_b[i]
            u = jax.nn.relu(_bn_rows(u, vn_bn1_g[i], vn_bn1_b[i]))
            u = u @ vn_lin2_W[i] + vn_lin2_b[i]
            u = jax.nn.relu(_bn_rows(u, vn_bn2_g[i], vn_bn2_b[i]))
            vn = vn + u
            hp = h + vn[batch]

    gemb = jax.ops.segment_sum(h, batch, num_segments=NUM_GRAPHS)
    logits = _cls(gemb, cls_W1, cls_b1.reshape(1, EMB // 2),
                  cls_W2, cls_b2.reshape(1, NUM_CLASSES))
    return logits, gemb
